# baseline (speedup 1.0000x reference)
import jax
import jax.numpy as jnp
from jax.experimental import pallas as pl

_EPS = 1e-5
_HI = jax.lax.Precision.HIGHEST
_F32 = jnp.float32


def _bn_grouped(y, g, be, ngroups):
    ch = g.shape[1]
    m = jnp.mean(y, axis=0, keepdims=True)
    q = jnp.mean(y * y, axis=0, keepdims=True)
    inv = 1.0 / ngroups
    mch = sum(m[:, i * ch:(i + 1) * ch] for i in range(ngroups)) * inv
    qch = sum(q[:, i * ch:(i + 1) * ch] for i in range(ngroups)) * inv
    v = qch - mch * mch
    scale = jax.lax.rsqrt(v + _EPS) * g
    shift = be - mch * scale
    if ngroups > 1:
        scale = jnp.concatenate([scale] * ngroups, axis=1)
        shift = jnp.concatenate([shift] * ngroups, axis=1)
    return y * scale + shift


def _mm_bn_relu_body(ngroups, p_ref, w_ref, g_ref, be_ref, o_ref):
    y = jnp.dot(p_ref[...], w_ref[...], preferred_element_type=_F32)
    o_ref[...] = jax.nn.relu(_bn_grouped(y, g_ref[...], be_ref[...], ngroups))


def _vq_body(h2_ref, w3_ref, g3_ref, be3_ref, emb_ref, embt_ref, w4_ref,
             g4_ref, be4_ref, ze_ref, zq_ref, d1_ref):
    n = h2_ref.shape[0]
    k = emb_ref.shape[0]
    z = jnp.dot(h2_ref[...], w3_ref[...], preferred_element_type=_F32)
    ze = _bn_grouped(z, g3_ref[...], be3_ref[...], 1)
    ze_ref[...] = ze

    embt = embt_ref[...]
    emb = emb_ref[...]
    w4 = w4_ref[...]
    e2 = jnp.sum(embt * embt, axis=0, keepdims=True)
    iota = jax.lax.broadcasted_iota(jnp.int32, (1, k), 1)

    nchunks = 8
    ch = n // nchunks
    zqs = []
    ys = []
    for c in range(nchunks):
        zc = ze[c * ch:(c + 1) * ch]
        d = e2 - 2.0 * jnp.dot(zc, embt, preferred_element_type=_F32,
                               precision=_HI)
        mn = jnp.min(d, axis=1, keepdims=True)
        masked = jnp.where(d == mn, iota, k)
        idx = jnp.min(masked, axis=1, keepdims=True)
        onehot = (iota == idx).astype(_F32)
        zq = jnp.dot(onehot, emb, preferred_element_type=_F32,
                     precision=_HI)
        zqs.append(zq)
        ys.append(jnp.dot(zq, w4, preferred_element_type=_F32,
                          precision=_HI))
    zq_ref[...] = jnp.concatenate(zqs, axis=0)
    y = jnp.concatenate(ys, axis=0)
    d1_ref[...] = jax.nn.relu(_bn_grouped(y, g4_ref[...], be4_ref[...], 1))


def _deconv6_body(p_ref, w_ref, b_ref, o_ref):
    y = jnp.dot(p_ref[...], w_ref[...], preferred_element_type=_F32)
    o_ref[...] = jax.nn.sigmoid(y + b_ref[...])


_TTAPS = {0: ((1, 1), (3, 0)), 1: ((0, 2), (2, 1))}


def _deconv_im2col(xp, h, cin):
    n = xp.shape[0] * h * h
    groups = []
    for a in range(2):
        for c in range(2):
            for kh, r0 in _TTAPS[a]:
                for kw, c0 in _TTAPS[c]:
                    groups.append(
                        xp[:, r0:r0 + h, c0:c0 + h, :].reshape(n, cin))
    return jnp.concatenate(groups, axis=-1)


def _deconv_weight(w):
    blocks = []
    for a in range(2):
        for c in range(2):
            blocks.append(jnp.concatenate(
                [w[:, :, kh, kw] for kh, _ in _TTAPS[a] for kw, _ in _TTAPS[c]],
                axis=0))
    return jax.scipy.linalg.block_diag(*blocks)


def kernel(x, w1, b1, g1, be1, w2, b2, g2, be2, w3, b3, g3, be3, emb,
           w4, b4, g4, be4, w5, b5, g5, be5, w6, b6):
    B = x.shape[0]
    K, C = emb.shape
    H1 = x.shape[2] // 2
    H2 = H1 // 2
    N1 = B * H1 * H1
    N2 = B * H2 * H2

    xp = jnp.pad(x[:, 0], ((0, 0), (1, 1), (1, 1)))
    p1 = jnp.stack([xp[:, kh:kh + 2 * H1 - 1:2, kw:kw + 2 * H1 - 1:2]
                    for kh in range(4) for kw in range(4)],
                   axis=-1).reshape(N1 // 8, 128)
    wk1 = jnp.transpose(w1.reshape(16, 16))
    w1bd = jax.scipy.linalg.block_diag(*([wk1] * 8))
    h1 = pl.pallas_call(
        lambda *a: _mm_bn_relu_body(8, *a),
        out_shape=jax.ShapeDtypeStruct((N1 // 8, 128), _F32),
    )(p1, w1bd, g1.reshape(1, 16), be1.reshape(1, 16))

    hp = jnp.pad(h1.reshape(B, H1, H1, 16), ((0, 0), (1, 1), (1, 1), (0, 0)))
    p2 = jnp.concatenate([hp[:, kh:kh + 2 * H2 - 1:2, kw:kw + 2 * H2 - 1:2, :]
                          for kh in range(4) for kw in range(4)],
                         axis=-1).reshape(N2, 256)
    wk2 = jnp.transpose(w2, (2, 3, 1, 0)).reshape(256, 32)
    h2 = pl.pallas_call(
        lambda *a: _mm_bn_relu_body(1, *a),
        out_shape=jax.ShapeDtypeStruct((N2, 32), _F32),
    )(p2, wk2, g2.reshape(1, 32), be2.reshape(1, 32))

    wk3 = jnp.transpose(w3.reshape(64, 32))
    wk4 = jnp.transpose(w4.reshape(32, 64))
    z_e, z_q, d1 = pl.pallas_call(
        _vq_body,
        out_shape=(jax.ShapeDtypeStruct((N2, C), _F32),
                   jax.ShapeDtypeStruct((N2, C), _F32),
                   jax.ShapeDtypeStruct((N2, 32), _F32)),
    )(h2, wk3, g3.reshape(1, 64), be3.reshape(1, 64), emb,
      jnp.transpose(emb), wk4, g4.reshape(1, 32), be4.reshape(1, 32))

    dp = jnp.pad(d1.reshape(B, H2, H2, 32), ((0, 0), (1, 1), (1, 1), (0, 0)))
    p5 = _deconv_im2col(dp, H2, 32)
    wk5 = _deconv_weight(w5)
    o5 = pl.pallas_call(
        lambda *a: _mm_bn_relu_body(4, *a),
        out_shape=jax.ShapeDtypeStruct((N2, 64), _F32),
    )(p5, wk5, g5.reshape(1, 16), be5.reshape(1, 16))
    h5 = (o5.reshape(B, H2, H2, 2, 2, 16)
          .transpose(0, 1, 3, 2, 4, 5)
          .reshape(B, H1, H1, 16))

    h5p = jnp.pad(h5, ((0, 0), (1, 1), (1, 1), (0, 0)))
    p6 = _deconv_im2col(h5p, H1, 16)
    wk6 = _deconv_weight(w6)
    nblk = 8
    blk = N1 // nblk
    o6 = pl.pallas_call(
        _deconv6_body,
        grid=(nblk,),
        in_specs=[pl.BlockSpec((blk, 256), lambda i: (i, 0)),
                  pl.BlockSpec((256, 4), lambda i: (0, 0)),
                  pl.BlockSpec((1, 1), lambda i: (0, 0))],
        out_specs=pl.BlockSpec((blk, 4), lambda i: (i, 0)),
        out_shape=jax.ShapeDtypeStruct((N1, 4), _F32),
    )(p6, wk6, b6.reshape(1, 1))
    x_tilde = (o6.reshape(B, H1, H1, 2, 2)
               .transpose(0, 1, 3, 2, 4)
               .reshape(B, 1, 2 * H1, 2 * H1))

    z_e_x = z_e.reshape(B, H2, H2, C).transpose(0, 3, 1, 2)
    z_q_x = z_q.reshape(B, H2, H2, C).transpose(0, 3, 1, 2)
    return (x_tilde, z_e_x, z_q_x)
